# baseline (speedup 1.0000x reference)
import functools

import jax
import jax.numpy as jnp
from jax import lax
from jax.experimental import pallas as pl
from jax.experimental.pallas import tpu as pltpu
from jax.experimental.pallas import tpu_sc as plsc

A_DIMS = 100000
BATCH = 16384
ROW = 64
RPB = 1024
TCOL = 128
CLAST = (A_DIMS // TCOL) * TCOL
PART = A_DIMS - CLAST
NFULL = A_DIMS // TCOL - 1
NC, NS, L = 2, 16, 16
NW = NC * NS
BPW = BATCH // NW
NCHUNK = 4
CHUNK = BPW // NCHUNK
NBUF = 7
DPW = BPW * ROW

_mesh = plsc.VectorSubcoreMesh(core_axis_name="c", subcore_axis_name="s")


@functools.partial(
    pl.kernel,
    out_type=jax.ShapeDtypeStruct((BATCH,), jnp.float32),
    mesh=_mesh,
    compiler_params=pltpu.CompilerParams(needs_layout_passes=False),
    scratch_types=[
        pltpu.VMEM((BPW + L,), jnp.int32),
        pltpu.VMEM((BPW + L,), jnp.int32),
        pltpu.VMEM((BPW + L,), jnp.int32),
        pltpu.VMEM((BPW + L,), jnp.int32),
        pltpu.VMEM((BPW + L,), jnp.int32),
        pltpu.VMEM((NCHUNK, CHUNK), jnp.int32),
        pltpu.VMEM((NCHUNK, CHUNK), jnp.int32),
        pltpu.VMEM((NBUF, ROW, TCOL), jnp.float32),
        pltpu.VMEM((RPB * PART,), jnp.float32),
        pltpu.VMEM((DPW,), jnp.float32),
        pltpu.VMEM((BPW,), jnp.float32),
        pltpu.VMEM((BPW,), jnp.float32),
        pltpu.SemaphoreType.DMA((NBUF,)),
        pltpu.SemaphoreType.DMA,
        pltpu.SemaphoreType.DMA,
        pltpu.SemaphoreType.DMA,
    ],
)
def _sc_kernel(a_hbm, r0_hbm, f_hbm, fr0_hbm, fa_hbm, perm_hbm, def_hbm,
               w_hbm, wtail_hbm, b_hbm, out_hbm,
               a_v, r0_v, f_v, fr0_v, fa_v, idx_v, pidx_v, blk_v, wt_v,
               d_v, bg_v, o_v, sem_w, sem_b, sem_in, sem_out):
    wid = lax.axis_index("s") * NC + lax.axis_index("c")
    base = wid * BPW

    for j in range(NCHUNK):
        pltpu.async_copy(
            a_hbm.at[pl.ds(base + j * CHUNK, CHUNK)], idx_v.at[j], sem_in)
        pltpu.async_copy(
            perm_hbm.at[pl.ds(base + j * CHUNK, CHUNK)], pidx_v.at[j], sem_in)
    pltpu.sync_copy(a_hbm.at[pl.ds(base, BPW)], a_v.at[pl.ds(0, BPW)])
    pltpu.sync_copy(r0_hbm.at[pl.ds(base, BPW)], r0_v.at[pl.ds(0, BPW)])
    pltpu.sync_copy(f_hbm.at[pl.ds(base, BPW)], f_v.at[pl.ds(0, BPW)])
    pltpu.sync_copy(fr0_hbm.at[pl.ds(base, BPW)], fr0_v.at[pl.ds(0, BPW)])
    pltpu.sync_copy(fa_hbm.at[pl.ds(base, BPW)], fa_v.at[pl.ds(0, BPW)])

    def sread(ref, e):
        return ref[pl.ds(e, L)][0]

    dcp = pltpu.async_copy(def_hbm.at[pl.ds(base * ROW, DPW)], d_v, sem_in)
    wtcp = pltpu.async_copy(wtail_hbm, wt_v, sem_in)
    for j in range(NCHUNK):
        pltpu.make_async_copy(
            a_hbm.at[pl.ds(0, CHUNK)], idx_v.at[j], sem_in).wait()
        pltpu.make_async_copy(
            a_hbm.at[pl.ds(0, CHUNK)], pidx_v.at[j], sem_in).wait()

    bcps = [
        pltpu.async_copy(
            b_hbm.at[idx_v.at[j]], bg_v.at[pl.ds(j * CHUNK, CHUNK)], sem_b)
        for j in range(NCHUNK)
    ]

    nfetch = sread(f_v, BPW - 1) + 1

    def fire(nf):
        r0 = pl.multiple_of(sread(fr0_v, nf), ROW)
        c0 = pl.multiple_of(
            jnp.minimum(sread(fa_v, nf) // TCOL, NFULL) * TCOL, TCOL)
        pltpu.async_copy(
            w_hbm.at[pl.ds(r0, ROW), pl.ds(c0, TCOL)],
            blk_v.at[nf % NBUF], sem_w.at[nf % NBUF])

    def wait_blk(df):
        pltpu.make_async_copy(
            w_hbm.at[pl.ds(0, ROW), pl.ds(0, TCOL)], blk_v.at[df % NBUF],
            sem_w.at[df % NBUF]).wait()

    lane = lax.iota(jnp.int32, L)

    def compute(e, buf, outv):
        ae = sread(a_v, e)
        dchunks = [d_v[pl.ds(e * ROW + k * L, L)] for k in range(ROW // L)]

        def from_blk():
            col = jnp.full((L,), ae % TCOL, jnp.int32)
            blk = blk_v.at[buf]
            acc = plsc.load_gather(blk, [lane, col]) * dchunks[0]
            for k in range(1, ROW // L):
                acc = acc + (plsc.load_gather(blk, [k * L + lane, col])
                             * dchunks[k])
            return acc

        def from_tail():
            idx0 = jnp.full((L,), sread(r0_v, e) * PART + (ae - CLAST),
                            jnp.int32)
            fidx = idx0 + lane * PART
            acc = plsc.load_gather(wt_v, [fidx]) * dchunks[0]
            for k in range(1, ROW // L):
                acc = acc + (plsc.load_gather(wt_v, [fidx + k * L * PART])
                             * dchunks[k])
            return acc

        acc = lax.cond(ae < CLAST, from_blk, from_tail)
        return jnp.where(lane == e % L, jnp.sum(acc), outv)

    for c in bcps:
        c.wait()
    dcp.wait()
    wtcp.wait()

    def body(e, carry):
        nextf, donef, outv = carry
        fe = sread(f_v, e)

        target = jnp.minimum(fe + NBUF, nfetch)

        def fire_body(nf):
            fire(nf)
            return nf + 1

        nextf = lax.while_loop(lambda nf: nf < target, fire_body, nextf)

        def wait_body(df):
            wait_blk(df)
            return df + 1

        donef = lax.while_loop(lambda df: df <= fe, wait_body, donef)

        outv = compute(e, fe % NBUF, outv)

        @pl.when(e % L == L - 1)
        def _():
            g = e - (L - 1)
            o_v[pl.ds(g, L)] = outv + bg_v[pl.ds(g, L)]

        return nextf, donef, outv

    lax.fori_loop(
        0, BPW, body,
        (jnp.int32(0), jnp.int32(0), jnp.zeros((L,), jnp.float32)))

    ocps = [
        pltpu.async_copy(
            o_v.at[pl.ds(j * CHUNK, CHUNK)], out_hbm.at[pidx_v.at[j]],
            sem_out)
        for j in range(NCHUNK)
    ]
    for c in ocps:
        c.wait()


def kernel(a, bh_pos, def_pos, W, b):
    a32 = a.astype(jnp.int32)
    r0 = (bh_pos[:, 0].astype(jnp.int32) * 4
          + bh_pos[:, 1].astype(jnp.int32)) * ROW
    nj = A_DIMS // TCOL + 1
    key = (r0 // ROW) * nj + a32 // TCOL
    packed = key * BATCH + jnp.arange(BATCH, dtype=jnp.int32)
    perm = packed % BATCH
    key_s = packed // BATCH
    a_s = a32[perm]
    r0_s = (key_s // nj) * ROW

    prev = jnp.concatenate([key_s[:1] - 1, key_s[:-1]])
    seg_start = (jnp.arange(BATCH, dtype=jnp.int32) % BPW) == 0
    nondup = ((key_s != prev) | seg_start).astype(jnp.int32)
    f2 = jnp.cumsum(nondup.reshape(NW, BPW), axis=1, dtype=jnp.int32) - 1
    f_flat = f2.reshape(BATCH)
    pos = (jnp.arange(BATCH, dtype=jnp.int32) // BPW) * BPW + f_flat
    fr0 = jnp.zeros((BATCH,), jnp.int32).at[pos].set(r0_s)
    fa = jnp.zeros((BATCH,), jnp.int32).at[pos].set(a_s)

    def_s = def_pos.astype(jnp.float32).reshape(BATCH, ROW)[perm].reshape(
        BATCH * ROW)
    wp = W.transpose(1, 2, 3, 4, 0).reshape(RPB, A_DIMS)
    wtail = wp[:, CLAST:].reshape(RPB * PART)
    return _sc_kernel(a_s, r0_s, f_flat, fr0, fa, perm, def_s, wp, wtail, b)


# speedup vs baseline: 2.2297x; 2.2297x over previous
import functools

import jax
import jax.numpy as jnp
from jax import lax
from jax.experimental import pallas as pl
from jax.experimental.pallas import tpu as pltpu
from jax.experimental.pallas import tpu_sc as plsc

A_DIMS = 100000
BATCH = 16384
ROW = 64
RPB = 1024
TCOL = 128
CLAST = (A_DIMS // TCOL) * TCOL
PART = A_DIMS - CLAST
NFULL = A_DIMS // TCOL - 1
NC, NS, L = 2, 16, 16
NW = NC * NS
BPW = BATCH // NW
NCHUNK = 4
CHUNK = BPW // NCHUNK
NBUF = 7
DPW = BPW * ROW

_mesh = plsc.VectorSubcoreMesh(core_axis_name="c", subcore_axis_name="s")


@functools.partial(
    pl.kernel,
    out_type=jax.ShapeDtypeStruct((BATCH,), jnp.float32),
    mesh=_mesh,
    compiler_params=pltpu.CompilerParams(needs_layout_passes=False),
    scratch_types=[
        pltpu.VMEM((BPW + L,), jnp.int32),
        pltpu.VMEM((BPW + L,), jnp.int32),
        pltpu.VMEM((NCHUNK, CHUNK), jnp.int32),
        pltpu.VMEM((NCHUNK, CHUNK), jnp.int32),
        pltpu.VMEM((NBUF, ROW, TCOL), jnp.float32),
        pltpu.VMEM((RPB * PART,), jnp.float32),
        pltpu.VMEM((DPW,), jnp.float32),
        pltpu.VMEM((BPW,), jnp.float32),
        pltpu.VMEM((BPW,), jnp.float32),
        pltpu.SemaphoreType.DMA((NBUF,)),
        pltpu.SemaphoreType.DMA,
        pltpu.SemaphoreType.DMA,
        pltpu.SemaphoreType.DMA,
    ],
)
def _sc_kernel(a_hbm, r0_hbm, perm_hbm, def_hbm, w_hbm, wtail_hbm, b_hbm,
               out_hbm,
               a_v, r0_v, idx_v, pidx_v, blk_v, wt_v, d_v, bg_v, o_v,
               sem_w, sem_b, sem_in, sem_out):
    wid = lax.axis_index("s") * NC + lax.axis_index("c")
    base = wid * BPW

    for j in range(NCHUNK):
        pltpu.async_copy(
            a_hbm.at[pl.ds(base + j * CHUNK, CHUNK)], idx_v.at[j], sem_in)
        pltpu.async_copy(
            perm_hbm.at[pl.ds(base + j * CHUNK, CHUNK)], pidx_v.at[j], sem_in)
    pltpu.sync_copy(a_hbm.at[pl.ds(base, BPW)], a_v.at[pl.ds(0, BPW)])
    pltpu.sync_copy(r0_hbm.at[pl.ds(base, BPW)], r0_v.at[pl.ds(0, BPW)])

    def sread(ref, e):
        return ref[pl.ds(e, L)][0]

    dcp = pltpu.async_copy(def_hbm.at[pl.ds(base * ROW, DPW)], d_v, sem_in)
    wtcp = pltpu.async_copy(wtail_hbm, wt_v, sem_in)
    for j in range(NCHUNK):
        pltpu.make_async_copy(
            a_hbm.at[pl.ds(0, CHUNK)], idx_v.at[j], sem_in).wait()
        pltpu.make_async_copy(
            a_hbm.at[pl.ds(0, CHUNK)], pidx_v.at[j], sem_in).wait()

    bcps = [
        pltpu.async_copy(
            b_hbm.at[idx_v.at[j]], bg_v.at[pl.ds(j * CHUNK, CHUNK)], sem_b)
        for j in range(NCHUNK)
    ]

    def is_new_block(e):
        em1 = jnp.maximum(e - 1, 0)
        same = ((sread(a_v, e) // TCOL == sread(a_v, em1) // TCOL)
                & (sread(r0_v, e) == sread(r0_v, em1)))
        return jnp.where(e == 0, jnp.int32(1), 1 - same.astype(jnp.int32))

    def fire(ew, nf):
        r0 = pl.multiple_of(sread(r0_v, ew), ROW)
        c0 = pl.multiple_of(
            jnp.minimum(sread(a_v, ew) // TCOL, NFULL) * TCOL, TCOL)
        pltpu.async_copy(
            w_hbm.at[pl.ds(r0, ROW), pl.ds(c0, TCOL)],
            blk_v.at[nf % NBUF], sem_w.at[nf % NBUF])

    def wait_blk(df):
        pltpu.make_async_copy(
            w_hbm.at[pl.ds(0, ROW), pl.ds(0, TCOL)], blk_v.at[df % NBUF],
            sem_w.at[df % NBUF]).wait()

    lane = lax.iota(jnp.int32, L)

    def compute(e, buf, outv):
        ae = sread(a_v, e)
        dchunks = [d_v[pl.ds(e * ROW + k * L, L)] for k in range(ROW // L)]

        def from_blk():
            col = jnp.full((L,), ae % TCOL, jnp.int32)
            blk = blk_v.at[buf]
            acc = plsc.load_gather(blk, [lane, col]) * dchunks[0]
            for k in range(1, ROW // L):
                acc = acc + (plsc.load_gather(blk, [k * L + lane, col])
                             * dchunks[k])
            return acc

        def from_tail():
            idx0 = jnp.full((L,), sread(r0_v, e) * PART + (ae - CLAST),
                            jnp.int32)
            fidx = idx0 + lane * PART
            acc = plsc.load_gather(wt_v, [fidx]) * dchunks[0]
            for k in range(1, ROW // L):
                acc = acc + (plsc.load_gather(wt_v, [fidx + k * L * PART])
                             * dchunks[k])
            return acc

        acc = lax.cond(ae < CLAST, from_blk, from_tail)
        return jnp.where(lane == e % L, jnp.sum(acc), outv)

    for c in bcps:
        c.wait()
    dcp.wait()
    wtcp.wait()

    def body(e, carry):
        ew, nf, curf, outv = carry
        nd = is_new_block(e)
        fe = curf + nd

        def fire_cond(c):
            cew, cnf = c
            return (cnf < fe + NBUF) & (cew < BPW)

        def fire_body(c):
            cew, cnf = c
            fnd = is_new_block(cew)

            @pl.when(fnd == 1)
            def _():
                fire(cew, cnf)

            return cew + 1, cnf + fnd

        ew, nf = lax.while_loop(fire_cond, fire_body, (ew, nf))

        @pl.when(nd == 1)
        def _():
            wait_blk(fe)

        outv = compute(e, fe % NBUF, outv)

        @pl.when(e % L == L - 1)
        def _():
            g = e - (L - 1)
            o_v[pl.ds(g, L)] = outv + bg_v[pl.ds(g, L)]

        return ew, nf, fe, outv

    lax.fori_loop(
        0, BPW, body,
        (jnp.int32(0), jnp.int32(0), jnp.int32(-1),
         jnp.zeros((L,), jnp.float32)))

    ocps = [
        pltpu.async_copy(
            o_v.at[pl.ds(j * CHUNK, CHUNK)], out_hbm.at[pidx_v.at[j]],
            sem_out)
        for j in range(NCHUNK)
    ]
    for c in ocps:
        c.wait()


def kernel(a, bh_pos, def_pos, W, b):
    a32 = a.astype(jnp.int32)
    r0 = (bh_pos[:, 0].astype(jnp.int32) * 4
          + bh_pos[:, 1].astype(jnp.int32)) * ROW
    nj = A_DIMS // TCOL + 1
    key = (r0 // ROW) * nj + a32 // TCOL
    packed = jnp.sort(key * BATCH + jnp.arange(BATCH, dtype=jnp.int32))
    perm = packed % BATCH
    key_s = packed // BATCH
    a_s = a32[perm]
    r0_s = (key_s // nj) * ROW

    def_s = def_pos.astype(jnp.float32).reshape(BATCH, ROW)[perm].reshape(
        BATCH * ROW)
    wp = W.transpose(1, 2, 3, 4, 0).reshape(RPB, A_DIMS)
    wtail = wp[:, CLAST:].reshape(RPB * PART)
    return _sc_kernel(a_s, r0_s, perm, def_s, wp, wtail, b)
